# baseline (speedup 1.0000x reference)
import functools
import jax
import jax.numpy as jnp
from jax import lax
from jax.experimental import pallas as pl
from jax.experimental.pallas import tpu as pltpu
from jax.experimental.pallas import tpu_sc as plsc

S, B, D = 4096, 4, 1024
NC, NS = 2, 16
NW = NC * NS
S_PER_W = S // NW
CH = 8
NCHUNK = S_PER_W // CH
NBUF = 3
NV = D // 16


def _sc_body(x_hbm, t_hbm, o_hbm, xb, tb, *sems):
    sin = sems[:NBUF]
    sout = sems[NBUF:]
    wid = lax.axis_index("s") * NC + lax.axis_index("c")
    base = wid * S_PER_W

    def start_in(ci, b):
        s0 = base + ci * CH
        pltpu.make_async_copy(x_hbm.at[pl.ds(s0, CH)], xb.at[b], sin[b]).start()
        pltpu.make_async_copy(t_hbm.at[pl.ds(s0, CH)], tb.at[b], sin[b]).start()

    def wait_in(b):
        pltpu.make_async_copy(x_hbm.at[pl.ds(0, CH)], xb.at[b], sin[b]).wait()
        pltpu.make_async_copy(t_hbm.at[pl.ds(0, CH)], tb.at[b], sin[b]).wait()

    def start_out(ci, b):
        dst = o_hbm.at[pl.ds(base + ci * CH, CH)]
        pltpu.make_async_copy(xb.at[b], dst, sout[b]).start()

    def wait_out(b):
        dst = o_hbm.at[pl.ds(base, CH)]
        pltpu.make_async_copy(xb.at[b], dst, sout[b]).wait()

    def compute(b):
        @plsc.parallel_loop(0, CH)
        def _(p):
            for v in range(NV):
                tv = tb[b, p, pl.ds(v * 16, 16)]
                for bb in range(B):
                    xb[b, p, bb, pl.ds(v * 16, 16)] += tv

    def step(ci, b, traced):
        bn = (b + 1) % NBUF
        if traced:
            @pl.when(ci >= NBUF - 1)
            def _():
                wait_out(bn)

            @pl.when(ci + 1 < NCHUNK)
            def _():
                start_in(ci + 1, bn)
        else:
            if ci >= NBUF - 1:
                wait_out(bn)
            if ci + 1 < NCHUNK:
                start_in(ci + 1, bn)
        wait_in(b)
        compute(b)
        start_out(ci, b)

    start_in(0, 0)

    NFULL = (NCHUNK // NBUF) * NBUF

    def group_body(g, carry):
        for b in range(NBUF):
            step(g * NBUF + b, b, True)
        return carry

    lax.fori_loop(0, NCHUNK // NBUF, group_body, 0)
    for ci in range(NFULL, NCHUNK):
        step(ci, ci % NBUF, False)
    for ci in range(max(NCHUNK - NBUF + 1, 0), NCHUNK):
        wait_out(ci % NBUF)


def kernel(x, table):
    mesh = plsc.VectorSubcoreMesh(core_axis_name="c", subcore_axis_name="s")
    f = functools.partial(
        pl.kernel,
        mesh=mesh,
        out_type=jax.ShapeDtypeStruct((S, B, D), jnp.float32),
        scratch_types=[
            pltpu.VMEM((NBUF, CH, B, D), jnp.float32),
            pltpu.VMEM((NBUF, CH, D), jnp.float32),
        ] + [pltpu.SemaphoreType.DMA] * (2 * NBUF),
    )(_sc_body)
    return f(x, table)


# speedup vs baseline: 1.0611x; 1.0611x over previous
import functools
import jax
import jax.numpy as jnp
from jax import lax
from jax.experimental import pallas as pl
from jax.experimental.pallas import tpu as pltpu
from jax.experimental.pallas import tpu_sc as plsc

S, B, D = 4096, 4, 1024
NC, NS = 2, 16
NW = NC * NS
S_PER_W = S // NW
CH = 8
NCHUNK = S_PER_W // CH
NBUF = 3
NV = D // 16


def _sc_body(x_hbm, t_hbm, o_hbm, xb, tb, *sems):
    sin = sems[:NBUF]
    sout = sems[NBUF:]
    wid = lax.axis_index("s") * NC + lax.axis_index("c")
    base = wid * S_PER_W

    def start_in(ci, b):
        s0 = base + ci * CH
        pltpu.make_async_copy(x_hbm.at[pl.ds(s0, CH)], xb.at[b], sin[b]).start()
        pltpu.make_async_copy(t_hbm.at[pl.ds(s0, CH)], tb.at[b], sin[b]).start()

    def wait_in(b):
        pltpu.make_async_copy(x_hbm.at[pl.ds(0, CH)], xb.at[b], sin[b]).wait()
        pltpu.make_async_copy(t_hbm.at[pl.ds(0, CH)], tb.at[b], sin[b]).wait()

    def wait_out(b):
        for _ in range(CH):
            pltpu.make_async_copy(xb.at[b, 0], o_hbm.at[base], sout[b]).wait()

    def compute(ci, b):
        def pos_body(p, c2):
            for v in range(NV):
                tv = tb[b, p, pl.ds(v * 16, 16)]
                for bb in range(B):
                    xb[b, p, bb, pl.ds(v * 16, 16)] += tv
            pltpu.make_async_copy(
                xb.at[b, p], o_hbm.at[base + ci * CH + p], sout[b]
            ).start()
            return c2

        lax.fori_loop(0, CH, pos_body, 0)

    def step(ci, b, traced):
        bn = (b + 1) % NBUF
        if traced:
            @pl.when(ci >= NBUF - 1)
            def _():
                wait_out(bn)

            @pl.when(ci + 1 < NCHUNK)
            def _():
                start_in(ci + 1, bn)
        else:
            if ci >= NBUF - 1:
                wait_out(bn)
            if ci + 1 < NCHUNK:
                start_in(ci + 1, bn)
        wait_in(b)
        compute(ci, b)

    start_in(0, 0)

    NFULL = (NCHUNK // NBUF) * NBUF

    def group_body(g, carry):
        for b in range(NBUF):
            step(g * NBUF + b, b, True)
        return carry

    lax.fori_loop(0, NCHUNK // NBUF, group_body, 0)
    for ci in range(NFULL, NCHUNK):
        step(ci, ci % NBUF, False)
    for ci in range(max(NCHUNK - NBUF + 1, 0), NCHUNK):
        wait_out(ci % NBUF)


def kernel(x, table):
    mesh = plsc.VectorSubcoreMesh(core_axis_name="c", subcore_axis_name="s")
    f = functools.partial(
        pl.kernel,
        mesh=mesh,
        out_type=jax.ShapeDtypeStruct((S, B, D), jnp.float32),
        scratch_types=[
            pltpu.VMEM((NBUF, CH, B, D), jnp.float32),
            pltpu.VMEM((NBUF, CH, D), jnp.float32),
        ] + [pltpu.SemaphoreType.DMA] * (2 * NBUF),
    )(_sc_body)
    return f(x, table)
